# baseline (speedup 1.0000x reference)
import functools

import jax
import jax.numpy as jnp
from jax.experimental import pallas as pl


def _leaky(v):
    return jnp.where(v >= 0, v, 0.01 * v)


def _mlp_body(feat_ref, W1_ref, b1_ref, W2_ref, b2_ref, W3_ref, b3_ref,
              We_ref, be_ref, Wc_ref, bc_ref, emb_ref, out_ref):
    h = _leaky(feat_ref[:])
    h = _leaky(jnp.dot(h, W1_ref[:], preferred_element_type=jnp.float32) + b1_ref[:])
    h = _leaky(jnp.dot(h, W2_ref[:], preferred_element_type=jnp.float32) + b2_ref[:])
    h = jnp.dot(h, W3_ref[:], preferred_element_type=jnp.float32) + b3_ref[:]
    e = jnp.dot(h, We_ref[:], preferred_element_type=jnp.float32) + be_ref[:]
    emb_ref[:] = e
    out_ref[:] = jnp.dot(e, Wc_ref[:], preferred_element_type=jnp.float32) + bc_ref[:]


@functools.partial(jax.jit, static_argnames=("block",))
def _mlp_head(feat, W1, b1, W2, b2, W3, b3, We, be, Wc, bc, block=2000):
    n, f = feat.shape
    grid = (n // block,)
    full = lambda *s: pl.BlockSpec(s, lambda i: tuple(0 for _ in s))
    return pl.pallas_call(
        _mlp_body,
        grid=grid,
        in_specs=[
            pl.BlockSpec((block, f), lambda i: (i, 0)),
            full(*W1.shape), full(*b1.shape),
            full(*W2.shape), full(*b2.shape),
            full(*W3.shape), full(*b3.shape),
            full(*We.shape), full(*be.shape),
            full(*Wc.shape), full(*bc.shape),
        ],
        out_specs=[
            pl.BlockSpec((block, We.shape[1]), lambda i: (i, 0)),
            pl.BlockSpec((block, Wc.shape[1]), lambda i: (i, 0)),
        ],
        out_shape=[
            jax.ShapeDtypeStruct((n, We.shape[1]), jnp.float32),
            jax.ShapeDtypeStruct((n, Wc.shape[1]), jnp.float32),
        ],
    )(feat, W1, b1, W2, b2, W3, b3, We, be, Wc, bc)


def _scatter_features(x, edge_index):
    n = x.shape[0]
    src = edge_index[0]
    dst = edge_index[1]
    deg = jnp.zeros((n,), jnp.float32).at[src].add(1.0)
    deg = jnp.maximum(deg, 1.0)
    inv_deg_src = (1.0 / deg)[src]

    def diffuse(h):
        msg = h[src] * inv_deg_src[:, None]
        agg = jnp.zeros_like(h).at[dst].add(msg)
        return 0.5 * (h + agg)

    def wavelets(h):
        powers = {}
        cur = h
        for k in range(1, 17):
            cur = diffuse(cur)
            if k in (1, 2, 4, 8, 16):
                powers[k] = cur
        return [powers[2 ** (j - 1)] - powers[2 ** j] for j in range(1, 5)]

    s1 = [jnp.abs(w) for w in wavelets(x)]
    u = jnp.concatenate(s1, axis=1)
    w2 = wavelets(u)
    f = x.shape[1]
    s2 = []
    for jp in range(2, 5):
        wj = jnp.abs(w2[jp - 1]).reshape(n, 4, f)
        for j in range(1, jp):
            s2.append(wj[:, j - 1, :])
    return jnp.concatenate([x] + s1 + s2, axis=1)


def kernel(x, edge_index, batch, W1, b1, W2, b2, W3, b3, We, be, Wc, bc):
    feat = _scatter_features(x, edge_index)
    emb, out = _mlp_head(feat, W1, b1, W2, b2, W3, b3, We, be, Wc, bc)
    return (emb, out)


# speedup vs baseline: 14.2638x; 14.2638x over previous
import functools

import jax
import jax.numpy as jnp
from jax import lax
from jax.experimental import pallas as pl
from jax.experimental.pallas import tpu as pltpu, tpu_sc as plsc

N = 50000
E = 1600000
EROWS = E // 128
NC = 2
NS = 16
NTILE = NC * NS
ROWS_PER_TILE = EROWS // NTILE
EXTRA_ROWS = EROWS - ROWS_PER_TILE * NTILE
CH = 10
NBLK = ROWS_PER_TILE // CH
NPT = N // NS


def _leaky(v):
    return jnp.where(v >= 0, v, 0.01 * v)


def _make_sc_pass(wp):
    mesh = plsc.VectorSubcoreMesh(core_axis_name="c", subcore_axis_name="s",
                                  num_cores=NC, num_subcores=NS)

    def body(g, src2, dst2, zer, part0, part1, agg_sh, idx_s, idx_d, rows, sem):
        c = lax.axis_index("c")
        s = lax.axis_index("s")
        w = s * NC + c
        pltpu.sync_copy(zer, agg_sh.at[pl.ds(s * NPT, NPT)])
        plsc.subcore_barrier()

        rb = w * ROWS_PER_TILE

        def blk(i, carry):
            base = rb + i * CH
            pltpu.sync_copy(src2.at[pl.ds(base, CH)], idx_s)
            pltpu.sync_copy(dst2.at[pl.ds(base, CH)], idx_d)
            for j in range(CH):
                pltpu.async_copy(g.at[idx_s.at[j]], rows, sem).wait()
                pltpu.sync_copy(rows, agg_sh.at[idx_d.at[j]], add=True)
            return carry

        lax.fori_loop(0, NBLK, blk, 0)

        @pl.when(w < EXTRA_ROWS)
        def _():
            er = ROWS_PER_TILE * NTILE + w
            pltpu.sync_copy(src2.at[pl.ds(er, 1)], idx_s.at[pl.ds(0, 1)])
            pltpu.sync_copy(dst2.at[pl.ds(er, 1)], idx_d.at[pl.ds(0, 1)])
            pltpu.async_copy(g.at[idx_s.at[0]], rows, sem).wait()
            pltpu.sync_copy(rows, agg_sh.at[idx_d.at[0]], add=True)

        plsc.subcore_barrier()
        sl = pl.ds(s * NPT, NPT)

        @pl.when(c == 0)
        def _():
            pltpu.sync_copy(agg_sh.at[sl], part0.at[sl])

        @pl.when(c == 1)
        def _():
            pltpu.sync_copy(agg_sh.at[sl], part1.at[sl])

    return pl.kernel(
        body,
        out_type=(jax.ShapeDtypeStruct((N, wp), jnp.float32),
                  jax.ShapeDtypeStruct((N, wp), jnp.float32)),
        mesh=mesh,
        scratch_types=[
            pltpu.VMEM_SHARED((N, wp), jnp.float32),
            pltpu.VMEM((CH, 128), jnp.int32),
            pltpu.VMEM((CH, 128), jnp.int32),
            pltpu.VMEM((128, wp), jnp.float32),
            pltpu.SemaphoreType.DMA,
        ],
        compiler_params=pltpu.CompilerParams(use_tc_tiling_on_sc=False),
    )


_BLK = 2000


def _full_spec(*shape):
    return pl.BlockSpec(shape, lambda i: tuple(0 for _ in shape))


def _row_spec(wp):
    return pl.BlockSpec((_BLK, wp), lambda i: (i, 0))


def _combine_body(h_ref, p0_ref, p1_ref, inv_ref, hn_ref, gn_ref):
    hn = 0.5 * h_ref[:] + 0.5 * (p0_ref[:] + p1_ref[:])
    hn_ref[:] = hn
    gn_ref[:] = hn * inv_ref[:]


def _combine(h, p0, p1, inv):
    wp = h.shape[1]
    return pl.pallas_call(
        _combine_body,
        grid=(N // _BLK,),
        in_specs=[_row_spec(wp), _row_spec(wp), _row_spec(wp), _row_spec(1)],
        out_specs=[_row_spec(wp), _row_spec(wp)],
        out_shape=[jax.ShapeDtypeStruct((N, wp), jnp.float32),
                   jax.ShapeDtypeStruct((N, wp), jnp.float32)],
    )(h, p0, p1, inv)


def _prep_body(x_ref, p0_ref, p1_ref, inv_ref, h0_ref, g0_ref):
    deg = p0_ref[:, 0:1] + p1_ref[:, 0:1]
    inv = 1.0 / jnp.maximum(deg, 1.0)
    inv_ref[:] = inv
    h0 = jnp.concatenate([x_ref[:], jnp.zeros((_BLK, 7), jnp.float32)], axis=1)
    h0_ref[:] = h0
    g0_ref[:] = h0 * inv


def _prep(x, p0, p1):
    return pl.pallas_call(
        _prep_body,
        grid=(N // _BLK,),
        in_specs=[_row_spec(9), _row_spec(16), _row_spec(16)],
        out_specs=[_row_spec(1), _row_spec(16), _row_spec(16)],
        out_shape=[jax.ShapeDtypeStruct((N, 1), jnp.float32),
                   jax.ShapeDtypeStruct((N, 16), jnp.float32),
                   jax.ShapeDtypeStruct((N, 16), jnp.float32)],
    )(x, p0, p1)


def _assemble_body(h1_ref, h2_ref, h4_ref, h8_ref, inv_ref, u_ref, gu_ref):
    b0 = jnp.abs(h1_ref[:] - h2_ref[:])[:, :9]
    b1 = jnp.abs(h2_ref[:] - h4_ref[:])[:, :9]
    b2 = jnp.abs(h4_ref[:] - h8_ref[:])[:, :9]
    u = jnp.concatenate([b0, b1, b2, jnp.zeros((_BLK, 5), jnp.float32)], axis=1)
    u_ref[:] = u
    gu_ref[:] = u * inv_ref[:]


def _assemble(h1, h2, h4, h8, inv):
    return pl.pallas_call(
        _assemble_body,
        grid=(N // _BLK,),
        in_specs=[_row_spec(16)] * 4 + [_row_spec(1)],
        out_specs=[_row_spec(32), _row_spec(32)],
        out_shape=[jax.ShapeDtypeStruct((N, 32), jnp.float32),
                   jax.ShapeDtypeStruct((N, 32), jnp.float32)],
    )(h1, h2, h4, h8, inv)


def _mlp_body(x_ref, h1_ref, h2_ref, h4_ref, h8_ref, h16_ref,
              u2_ref, u4_ref, u8_ref, u16_ref,
              W1_ref, b1_ref, W2_ref, b2_ref, W3_ref, b3_ref,
              We_ref, be_ref, Wc_ref, bc_ref, emb_ref, out_ref):
    s1_1 = jnp.abs(h1_ref[:] - h2_ref[:])[:, :9]
    s1_2 = jnp.abs(h2_ref[:] - h4_ref[:])[:, :9]
    s1_3 = jnp.abs(h4_ref[:] - h8_ref[:])[:, :9]
    s1_4 = jnp.abs(h8_ref[:] - h16_ref[:])[:, :9]
    d24 = jnp.abs(u2_ref[:] - u4_ref[:])
    d48 = jnp.abs(u4_ref[:] - u8_ref[:])
    d816 = jnp.abs(u8_ref[:] - u16_ref[:])
    feat = jnp.concatenate([
        x_ref[:], s1_1, s1_2, s1_3, s1_4,
        d24[:, 0:9],
        d48[:, 0:9], d48[:, 9:18],
        d816[:, 0:9], d816[:, 9:18], d816[:, 18:27],
    ], axis=1)
    h = _leaky(feat)
    h = _leaky(jnp.dot(h, W1_ref[:], preferred_element_type=jnp.float32) + b1_ref[:])
    h = _leaky(jnp.dot(h, W2_ref[:], preferred_element_type=jnp.float32) + b2_ref[:])
    h = jnp.dot(h, W3_ref[:], preferred_element_type=jnp.float32) + b3_ref[:]
    e = jnp.dot(h, We_ref[:], preferred_element_type=jnp.float32) + be_ref[:]
    emb_ref[:] = e
    out_ref[:] = jnp.dot(e, Wc_ref[:], preferred_element_type=jnp.float32) + bc_ref[:]


def _mlp(x, h1, h2, h4, h8, h16, u2, u4, u8, u16,
         W1, b1, W2, b2, W3, b3, We, be, Wc, bc):
    weight_specs = [_full_spec(*a.shape)
                    for a in (W1, b1, W2, b2, W3, b3, We, be, Wc, bc)]
    return pl.pallas_call(
        _mlp_body,
        grid=(N // _BLK,),
        in_specs=([_row_spec(9)] + [_row_spec(16)] * 5 + [_row_spec(32)] * 4
                  + weight_specs),
        out_specs=[_row_spec(32), _row_spec(1)],
        out_shape=[jax.ShapeDtypeStruct((N, 32), jnp.float32),
                   jax.ShapeDtypeStruct((N, 1), jnp.float32)],
    )(x, h1, h2, h4, h8, h16, u2, u4, u8, u16,
      W1, b1, W2, b2, W3, b3, We, be, Wc, bc)


def kernel(x, edge_index, batch, W1, b1, W2, b2, W3, b3, We, be, Wc, bc):
    src2 = edge_index[0].reshape(EROWS, 128)
    dst2 = edge_index[1].reshape(EROWS, 128)
    zer16 = jnp.zeros((NPT, 16), jnp.float32)
    zer32 = jnp.zeros((NPT, 32), jnp.float32)
    ones16 = jnp.ones((N, 16), jnp.float32)

    sc16 = _make_sc_pass(16)
    sc32 = _make_sc_pass(32)

    d0, d1 = sc16(ones16, src2, src2, zer16)
    inv, h, g = _prep(x, d0, d1)

    snaps1 = {}
    for k in range(1, 17):
        p0, p1 = sc16(g, src2, dst2, zer16)
        h, g = _combine(h, p0, p1, inv)
        if k in (1, 2, 4, 8, 16):
            snaps1[k] = h

    u, gu = _assemble(snaps1[1], snaps1[2], snaps1[4], snaps1[8], inv)

    snaps2 = {}
    h2s, g2s = u, gu
    for k in range(1, 17):
        p0, p1 = sc32(g2s, src2, dst2, zer32)
        h2s, g2s = _combine(h2s, p0, p1, inv)
        if k in (2, 4, 8, 16):
            snaps2[k] = h2s

    emb, out = _mlp(x, snaps1[1], snaps1[2], snaps1[4], snaps1[8], snaps1[16],
                    snaps2[2], snaps2[4], snaps2[8], snaps2[16],
                    W1, b1, W2, b2, W3, b3, We, be, Wc, bc)
    return (emb, out)


# speedup vs baseline: 24.5680x; 1.7224x over previous
import functools

import jax
import jax.numpy as jnp
from jax import lax
from jax.experimental import pallas as pl
from jax.experimental.pallas import tpu as pltpu, tpu_sc as plsc

N = 50000
E = 1600000
NC = 2
NS = 16
NTILE = NC * NS
ROWS_PER_TILE = 392
EROWS = ROWS_PER_TILE * NTILE
EPAD = EROWS * 128 - E
NROWS = 50048
NPT = NROWS // NS
D = 7
NG = ROWS_PER_TILE // D


def _leaky(v):
    return jnp.where(v >= 0, v, 0.01 * v)


def _make_sc_pass(wp):
    mesh = plsc.VectorSubcoreMesh(core_axis_name="c", subcore_axis_name="s",
                                  num_cores=NC, num_subcores=NS)

    def body(g, src2, dst2, zer, part0, part1, agg_sh, ibs, ibd, rows, *sems):
        gs = sems[:D]
        ss = sems[D:2 * D]
        is_ = sems[2 * D:2 * D + 2]
        id_ = sems[2 * D + 2:2 * D + 4]
        c = lax.axis_index("c")
        s = lax.axis_index("s")
        w = s * NC + c
        pltpu.sync_copy(zer, agg_sh.at[pl.ds(s * NPT, NPT)])
        plsc.subcore_barrier()

        rb = w * ROWS_PER_TILE

        def rslot(d):
            return rows.at[pl.ds(d * 128, 128)]

        def idx_src(blk):
            return src2.at[pl.ds(rb + blk * D, D)]

        def idx_dst(blk):
            return dst2.at[pl.ds(rb + blk * D, D)]

        pltpu.sync_copy(idx_src(0), ibs.at[0])
        pltpu.sync_copy(idx_dst(0), ibd.at[0])
        pltpu.async_copy(idx_src(1), ibs.at[1], is_[1])
        pltpu.async_copy(idx_dst(1), ibd.at[1], id_[1])
        for d in range(D):
            pltpu.async_copy(g.at[ibs.at[0].at[d]], rslot(d), gs[d])

        def pair(i, carry):
            for p in (0, 1):
                gi = i * 2 + p
                for d in range(D):
                    pltpu.make_async_copy(g.at[ibs.at[p].at[d]],
                                          rslot(d), gs[d]).wait()
                    pltpu.async_copy(rslot(d), agg_sh.at[ibd.at[p].at[d]],
                                     ss[d], add=True)

                def gather_next():
                    pltpu.make_async_copy(idx_src(0), ibs.at[1 - p],
                                          is_[1 - p]).wait()
                    pltpu.make_async_copy(idx_dst(0), ibd.at[1 - p],
                                          id_[1 - p]).wait()
                    for d in range(D):
                        pltpu.make_async_copy(rslot(d),
                                              agg_sh.at[pl.ds(0, 128)],
                                              ss[d]).wait()
                        pltpu.async_copy(g.at[ibs.at[1 - p].at[d]],
                                         rslot(d), gs[d])

                if p == 0:
                    gather_next()

                    @pl.when(i < NG // 2 - 1)
                    def _():
                        pltpu.async_copy(idx_src(gi + 2), ibs.at[0], is_[0])
                        pltpu.async_copy(idx_dst(gi + 2), ibd.at[0], id_[0])
                else:
                    @pl.when(i < NG // 2 - 1)
                    def _():
                        gather_next()
                        pltpu.async_copy(idx_src(gi + 2), ibs.at[1], is_[1])
                        pltpu.async_copy(idx_dst(gi + 2), ibd.at[1], id_[1])
            return carry

        lax.fori_loop(0, NG // 2, pair, 0)
        for d in range(D):
            pltpu.make_async_copy(rslot(d), agg_sh.at[pl.ds(0, 128)],
                                  ss[d]).wait()

        plsc.subcore_barrier()
        sl = pl.ds(s * NPT, NPT)

        @pl.when(c == 0)
        def _():
            pltpu.sync_copy(agg_sh.at[sl], part0.at[sl])

        @pl.when(c == 1)
        def _():
            pltpu.sync_copy(agg_sh.at[sl], part1.at[sl])

    return pl.kernel(
        body,
        out_type=(jax.ShapeDtypeStruct((NROWS, wp), jnp.float32),
                  jax.ShapeDtypeStruct((NROWS, wp), jnp.float32)),
        mesh=mesh,
        scratch_types=[
            pltpu.VMEM_SHARED((NROWS, wp), jnp.float32),
            pltpu.VMEM((2, D, 128), jnp.int32),
            pltpu.VMEM((2, D, 128), jnp.int32),
            pltpu.VMEM((D * 128, wp), jnp.float32),
        ] + [pltpu.SemaphoreType.DMA] * (2 * D + 4),
        compiler_params=pltpu.CompilerParams(use_tc_tiling_on_sc=False),
    )


_BLK = 2000


def _full_spec(*shape):
    return pl.BlockSpec(shape, lambda i: tuple(0 for _ in shape))


def _row_spec(wp):
    return pl.BlockSpec((_BLK, wp), lambda i: (i, 0))


def _combine_body(h_ref, p0_ref, p1_ref, inv_ref, hn_ref, gn_ref):
    hn = 0.5 * h_ref[:] + 0.5 * (p0_ref[:] + p1_ref[:])
    hn_ref[:] = hn
    gn_ref[:] = hn * inv_ref[:]


def _combine(h, p0, p1, inv):
    wp = h.shape[1]
    return pl.pallas_call(
        _combine_body,
        grid=(N // _BLK,),
        in_specs=[_row_spec(wp), _row_spec(wp), _row_spec(wp), _row_spec(1)],
        out_specs=[_row_spec(wp), _row_spec(wp)],
        out_shape=[jax.ShapeDtypeStruct((N, wp), jnp.float32),
                   jax.ShapeDtypeStruct((NROWS, wp), jnp.float32)],
    )(h, p0, p1, inv)


def _combine2_body(h_ref, p0a_ref, p1a_ref, p0b_ref, p1b_ref, inv_ref,
                   hn_ref, ga_ref, gb_ref):
    agg = jnp.concatenate([p0a_ref[:] + p1a_ref[:], p0b_ref[:] + p1b_ref[:]],
                          axis=1)
    hn = 0.5 * h_ref[:] + 0.5 * agg
    hn_ref[:] = hn
    inv = inv_ref[:]
    ga_ref[:] = hn[:, :16] * inv
    gb_ref[:] = hn[:, 16:] * inv


def _combine2(h, p0a, p1a, p0b, p1b, inv):
    return pl.pallas_call(
        _combine2_body,
        grid=(N // _BLK,),
        in_specs=[_row_spec(32)] + [_row_spec(16)] * 4 + [_row_spec(1)],
        out_specs=[_row_spec(32), _row_spec(16), _row_spec(16)],
        out_shape=[jax.ShapeDtypeStruct((N, 32), jnp.float32),
                   jax.ShapeDtypeStruct((NROWS, 16), jnp.float32),
                   jax.ShapeDtypeStruct((NROWS, 16), jnp.float32)],
    )(h, p0a, p1a, p0b, p1b, inv)


def _prep_body(x_ref, p0_ref, p1_ref, inv_ref, h0_ref, g0_ref):
    deg = p0_ref[:, 0:1] + p1_ref[:, 0:1]
    inv = 1.0 / jnp.maximum(deg, 1.0)
    inv_ref[:] = inv
    h0 = jnp.concatenate([x_ref[:], jnp.zeros((_BLK, 7), jnp.float32)], axis=1)
    h0_ref[:] = h0
    g0_ref[:] = h0 * inv


def _prep(x, p0, p1):
    return pl.pallas_call(
        _prep_body,
        grid=(N // _BLK,),
        in_specs=[_row_spec(9), _row_spec(16), _row_spec(16)],
        out_specs=[_row_spec(1), _row_spec(16), _row_spec(16)],
        out_shape=[jax.ShapeDtypeStruct((N, 1), jnp.float32),
                   jax.ShapeDtypeStruct((N, 16), jnp.float32),
                   jax.ShapeDtypeStruct((NROWS, 16), jnp.float32)],
    )(x, p0, p1)


def _assemble_body(h1_ref, h2_ref, h4_ref, h8_ref, inv_ref,
                   u_ref, ga_ref, gb_ref):
    b0 = jnp.abs(h1_ref[:] - h2_ref[:])[:, :9]
    b1 = jnp.abs(h2_ref[:] - h4_ref[:])[:, :9]
    b2 = jnp.abs(h4_ref[:] - h8_ref[:])[:, :9]
    u = jnp.concatenate([b0, b1, b2, jnp.zeros((_BLK, 5), jnp.float32)], axis=1)
    u_ref[:] = u
    inv = inv_ref[:]
    ga_ref[:] = u[:, :16] * inv
    gb_ref[:] = u[:, 16:] * inv


def _assemble(h1, h2, h4, h8, inv):
    return pl.pallas_call(
        _assemble_body,
        grid=(N // _BLK,),
        in_specs=[_row_spec(16)] * 4 + [_row_spec(1)],
        out_specs=[_row_spec(32), _row_spec(16), _row_spec(16)],
        out_shape=[jax.ShapeDtypeStruct((N, 32), jnp.float32),
                   jax.ShapeDtypeStruct((NROWS, 16), jnp.float32),
                   jax.ShapeDtypeStruct((NROWS, 16), jnp.float32)],
    )(h1, h2, h4, h8, inv)


def _mlp_body(x_ref, h1_ref, h2_ref, h4_ref, h8_ref, h16_ref,
              u2_ref, u4_ref, u8_ref, u16_ref,
              W1_ref, b1_ref, W2_ref, b2_ref, W3_ref, b3_ref,
              We_ref, be_ref, Wc_ref, bc_ref, emb_ref, out_ref):
    s1_1 = jnp.abs(h1_ref[:] - h2_ref[:])[:, :9]
    s1_2 = jnp.abs(h2_ref[:] - h4_ref[:])[:, :9]
    s1_3 = jnp.abs(h4_ref[:] - h8_ref[:])[:, :9]
    s1_4 = jnp.abs(h8_ref[:] - h16_ref[:])[:, :9]
    d24 = jnp.abs(u2_ref[:] - u4_ref[:])
    d48 = jnp.abs(u4_ref[:] - u8_ref[:])
    d816 = jnp.abs(u8_ref[:] - u16_ref[:])
    feat = jnp.concatenate([
        x_ref[:], s1_1, s1_2, s1_3, s1_4,
        d24[:, 0:9],
        d48[:, 0:9], d48[:, 9:18],
        d816[:, 0:9], d816[:, 9:18], d816[:, 18:27],
    ], axis=1)
    h = _leaky(feat)
    h = _leaky(jnp.dot(h, W1_ref[:], preferred_element_type=jnp.float32) + b1_ref[:])
    h = _leaky(jnp.dot(h, W2_ref[:], preferred_element_type=jnp.float32) + b2_ref[:])
    h = jnp.dot(h, W3_ref[:], preferred_element_type=jnp.float32) + b3_ref[:]
    e = jnp.dot(h, We_ref[:], preferred_element_type=jnp.float32) + be_ref[:]
    emb_ref[:] = e
    out_ref[:] = jnp.dot(e, Wc_ref[:], preferred_element_type=jnp.float32) + bc_ref[:]


def _mlp(x, h1, h2, h4, h8, h16, u2, u4, u8, u16,
         W1, b1, W2, b2, W3, b3, We, be, Wc, bc):
    weight_specs = [_full_spec(*a.shape)
                    for a in (W1, b1, W2, b2, W3, b3, We, be, Wc, bc)]
    return pl.pallas_call(
        _mlp_body,
        grid=(N // _BLK,),
        in_specs=([_row_spec(9)] + [_row_spec(16)] * 5 + [_row_spec(32)] * 4
                  + weight_specs),
        out_specs=[_row_spec(32), _row_spec(1)],
        out_shape=[jax.ShapeDtypeStruct((N, 32), jnp.float32),
                   jax.ShapeDtypeStruct((N, 1), jnp.float32)],
    )(x, h1, h2, h4, h8, h16, u2, u4, u8, u16,
      W1, b1, W2, b2, W3, b3, We, be, Wc, bc)


def kernel(x, edge_index, batch, W1, b1, W2, b2, W3, b3, We, be, Wc, bc):
    pad = jnp.full((EPAD,), N, jnp.int32)
    src2 = jnp.concatenate([edge_index[0], pad]).reshape(EROWS, 128)
    dst2 = jnp.concatenate([edge_index[1], pad]).reshape(EROWS, 128)
    zer16 = jnp.zeros((NPT, 16), jnp.float32)
    ones16 = jnp.ones((NROWS, 16), jnp.float32)

    sc16 = _make_sc_pass(16)

    d0, d1 = sc16(ones16, src2, src2, zer16)
    inv, h, g = _prep(x, d0, d1)

    snaps1 = {}
    for k in range(1, 17):
        p0, p1 = sc16(g, src2, dst2, zer16)
        h, g = _combine(h, p0, p1, inv)
        if k in (1, 2, 4, 8, 16):
            snaps1[k] = h

    u, ga, gb = _assemble(snaps1[1], snaps1[2], snaps1[4], snaps1[8], inv)

    snaps2 = {}
    h2s = u
    for k in range(1, 17):
        p0a, p1a = sc16(ga, src2, dst2, zer16)
        p0b, p1b = sc16(gb, src2, dst2, zer16)
        h2s, ga, gb = _combine2(h2s, p0a, p1a, p0b, p1b, inv)
        if k in (2, 4, 8, 16):
            snaps2[k] = h2s

    emb, out = _mlp(x, snaps1[1], snaps1[2], snaps1[4], snaps1[8], snaps1[16],
                    snaps2[2], snaps2[4], snaps2[8], snaps2[16],
                    W1, b1, W2, b2, W3, b3, We, be, Wc, bc)
    return (emb, out)


# speedup vs baseline: 26.3160x; 1.0712x over previous
import functools

import jax
import jax.numpy as jnp
from jax import lax
from jax.experimental import pallas as pl
from jax.experimental.pallas import tpu as pltpu, tpu_sc as plsc

N = 50000
E = 1600000
NC = 2
NS = 16
NTILE = NC * NS
ROWS_PER_TILE = 392
EROWS = ROWS_PER_TILE * NTILE
EPAD = EROWS * 128 - E
NROWS = 50048
NPT = NROWS // NS
D = 7
NG = ROWS_PER_TILE // D


def _leaky(v):
    return jnp.where(v >= 0, v, 0.01 * v)


def _make_sc_pass(wp):
    mesh = plsc.VectorSubcoreMesh(core_axis_name="c", subcore_axis_name="s",
                                  num_cores=NC, num_subcores=NS)

    def body(g, src2, dst2, zer, part0, part1, agg_sh, ibs, ibd, rows, *sems):
        gs = sems[:D]
        ss = sems[D:2 * D]
        is_ = sems[2 * D:2 * D + 2]
        id_ = sems[2 * D + 2:2 * D + 4]
        c = lax.axis_index("c")
        s = lax.axis_index("s")
        w = s * NC + c
        pltpu.sync_copy(zer, agg_sh.at[pl.ds(s * NPT, NPT)])
        plsc.subcore_barrier()

        rb = w * ROWS_PER_TILE

        def rslot(d):
            return rows.at[pl.ds(d * 128, 128)]

        def idx_src(blk):
            return src2.at[pl.ds(rb + blk * D, D)]

        def idx_dst(blk):
            return dst2.at[pl.ds(rb + blk * D, D)]

        pltpu.sync_copy(idx_src(0), ibs.at[0])
        pltpu.sync_copy(idx_dst(0), ibd.at[0])
        pltpu.async_copy(idx_src(1), ibs.at[1], is_[1])
        pltpu.async_copy(idx_dst(1), ibd.at[1], id_[1])
        for d in range(D):
            pltpu.async_copy(g.at[ibs.at[0].at[d]], rslot(d), gs[d])

        def pair(i, carry):
            for p in (0, 1):
                gi = i * 2 + p
                for d in range(D):
                    pltpu.make_async_copy(g.at[ibs.at[p].at[d]],
                                          rslot(d), gs[d]).wait()
                    pltpu.async_copy(rslot(d), agg_sh.at[ibd.at[p].at[d]],
                                     ss[d], add=True)

                def gather_next():
                    pltpu.make_async_copy(idx_src(0), ibs.at[1 - p],
                                          is_[1 - p]).wait()
                    pltpu.make_async_copy(idx_dst(0), ibd.at[1 - p],
                                          id_[1 - p]).wait()
                    for d in range(D):
                        pltpu.make_async_copy(rslot(d),
                                              agg_sh.at[pl.ds(0, 128)],
                                              ss[d]).wait()
                        pltpu.async_copy(g.at[ibs.at[1 - p].at[d]],
                                         rslot(d), gs[d])

                if p == 0:
                    gather_next()

                    @pl.when(i < NG // 2 - 1)
                    def _():
                        pltpu.async_copy(idx_src(gi + 2), ibs.at[0], is_[0])
                        pltpu.async_copy(idx_dst(gi + 2), ibd.at[0], id_[0])
                else:
                    @pl.when(i < NG // 2 - 1)
                    def _():
                        gather_next()
                        pltpu.async_copy(idx_src(gi + 2), ibs.at[1], is_[1])
                        pltpu.async_copy(idx_dst(gi + 2), ibd.at[1], id_[1])
            return carry

        lax.fori_loop(0, NG // 2, pair, 0)
        for d in range(D):
            pltpu.make_async_copy(rslot(d), agg_sh.at[pl.ds(0, 128)],
                                  ss[d]).wait()

        plsc.subcore_barrier()
        sl = pl.ds(s * NPT, NPT)

        @pl.when(c == 0)
        def _():
            pltpu.sync_copy(agg_sh.at[sl], part0.at[sl])

        @pl.when(c == 1)
        def _():
            pltpu.sync_copy(agg_sh.at[sl], part1.at[sl])

    return pl.kernel(
        body,
        out_type=(jax.ShapeDtypeStruct((NROWS, wp), jnp.float32),
                  jax.ShapeDtypeStruct((NROWS, wp), jnp.float32)),
        mesh=mesh,
        scratch_types=[
            pltpu.VMEM_SHARED((NROWS, wp), jnp.float32),
            pltpu.VMEM((2, D, 128), jnp.int32),
            pltpu.VMEM((2, D, 128), jnp.int32),
            pltpu.VMEM((D * 128, wp), jnp.float32),
        ] + [pltpu.SemaphoreType.DMA] * (2 * D + 4),
        compiler_params=pltpu.CompilerParams(use_tc_tiling_on_sc=False),
    )


_BLK = 2000


def _full_spec(*shape):
    return pl.BlockSpec(shape, lambda i: tuple(0 for _ in shape))


def _row_spec(wp):
    return pl.BlockSpec((_BLK, wp), lambda i: (i, 0))


def _combine_body(s_ref, p0_ref, p1_ref, inv_ref, sn_ref):
    sn_ref[:] = 0.5 * s_ref[:] + (0.5 * inv_ref[:]) * (p0_ref[:] + p1_ref[:])


def _combine(s, p0, p1, inv):
    return pl.pallas_call(
        _combine_body,
        grid=(N // _BLK,),
        in_specs=[_row_spec(16), _row_spec(16), _row_spec(16), _row_spec(1)],
        out_specs=[_row_spec(16)],
        out_shape=[jax.ShapeDtypeStruct((NROWS, 16), jnp.float32)],
    )(s, p0, p1, inv)[0]


def _prep_body(x_ref, p0_ref, p1_ref, inv_ref, deg_ref, s0_ref):
    deg = jnp.maximum(p0_ref[:, 0:1] + p1_ref[:, 0:1], 1.0)
    inv = 1.0 / deg
    inv_ref[:] = inv
    deg_ref[:] = deg
    h0 = jnp.concatenate([x_ref[:], jnp.zeros((_BLK, 7), jnp.float32)], axis=1)
    s0_ref[:] = h0 * inv


def _prep(x, p0, p1):
    return pl.pallas_call(
        _prep_body,
        grid=(N // _BLK,),
        in_specs=[_row_spec(9), _row_spec(16), _row_spec(16)],
        out_specs=[_row_spec(1), _row_spec(1), _row_spec(16)],
        out_shape=[jax.ShapeDtypeStruct((N, 1), jnp.float32),
                   jax.ShapeDtypeStruct((N, 1), jnp.float32),
                   jax.ShapeDtypeStruct((NROWS, 16), jnp.float32)],
    )(x, p0, p1)


def _assemble_body(s1_ref, s2_ref, s4_ref, s8_ref, ta_ref, tb_ref):
    b0 = jnp.abs(s1_ref[:] - s2_ref[:])[:, :9]
    b1 = jnp.abs(s2_ref[:] - s4_ref[:])[:, :9]
    b2 = jnp.abs(s4_ref[:] - s8_ref[:])[:, :9]
    t0 = jnp.concatenate([b0, b1, b2, jnp.zeros((_BLK, 5), jnp.float32)],
                         axis=1)
    ta_ref[:] = t0[:, :16]
    tb_ref[:] = t0[:, 16:]


def _assemble(s1, s2, s4, s8):
    return pl.pallas_call(
        _assemble_body,
        grid=(N // _BLK,),
        in_specs=[_row_spec(16)] * 4,
        out_specs=[_row_spec(16), _row_spec(16)],
        out_shape=[jax.ShapeDtypeStruct((NROWS, 16), jnp.float32),
                   jax.ShapeDtypeStruct((NROWS, 16), jnp.float32)],
    )(s1, s2, s4, s8)


def _mlp_body(x_ref, s1_ref, s2_ref, s4_ref, s8_ref, s16_ref,
              ta2_ref, tb2_ref, ta4_ref, tb4_ref,
              ta8_ref, tb8_ref, ta16_ref, tb16_ref, deg_ref,
              W1_ref, b1_ref, W2_ref, b2_ref, W3_ref, b3_ref,
              We_ref, be_ref, Wc_ref, bc_ref, emb_ref, out_ref):
    deg = deg_ref[:]
    s1_1 = (deg * jnp.abs(s1_ref[:] - s2_ref[:]))[:, :9]
    s1_2 = (deg * jnp.abs(s2_ref[:] - s4_ref[:]))[:, :9]
    s1_3 = (deg * jnp.abs(s4_ref[:] - s8_ref[:]))[:, :9]
    s1_4 = (deg * jnp.abs(s8_ref[:] - s16_ref[:]))[:, :9]
    d24 = deg * jnp.concatenate(
        [jnp.abs(ta2_ref[:] - ta4_ref[:]), jnp.abs(tb2_ref[:] - tb4_ref[:])],
        axis=1)
    d48 = deg * jnp.concatenate(
        [jnp.abs(ta4_ref[:] - ta8_ref[:]), jnp.abs(tb4_ref[:] - tb8_ref[:])],
        axis=1)
    d816 = deg * jnp.concatenate(
        [jnp.abs(ta8_ref[:] - ta16_ref[:]), jnp.abs(tb8_ref[:] - tb16_ref[:])],
        axis=1)
    feat = jnp.concatenate([
        x_ref[:], s1_1, s1_2, s1_3, s1_4,
        d24[:, 0:9],
        d48[:, 0:9], d48[:, 9:18],
        d816[:, 0:9], d816[:, 9:18], d816[:, 18:27],
    ], axis=1)
    h = _leaky(feat)
    h = _leaky(jnp.dot(h, W1_ref[:], preferred_element_type=jnp.float32) + b1_ref[:])
    h = _leaky(jnp.dot(h, W2_ref[:], preferred_element_type=jnp.float32) + b2_ref[:])
    h = jnp.dot(h, W3_ref[:], preferred_element_type=jnp.float32) + b3_ref[:]
    e = jnp.dot(h, We_ref[:], preferred_element_type=jnp.float32) + be_ref[:]
    emb_ref[:] = e
    out_ref[:] = jnp.dot(e, Wc_ref[:], preferred_element_type=jnp.float32) + bc_ref[:]


def _mlp(x, s1, s2, s4, s8, s16, t2, t4, t8, t16, deg,
         W1, b1, W2, b2, W3, b3, We, be, Wc, bc):
    weight_specs = [_full_spec(*a.shape)
                    for a in (W1, b1, W2, b2, W3, b3, We, be, Wc, bc)]
    return pl.pallas_call(
        _mlp_body,
        grid=(N // _BLK,),
        in_specs=([_row_spec(9)] + [_row_spec(16)] * 13 + [_row_spec(1)]
                  + weight_specs),
        out_specs=[_row_spec(32), _row_spec(1)],
        out_shape=[jax.ShapeDtypeStruct((N, 32), jnp.float32),
                   jax.ShapeDtypeStruct((N, 1), jnp.float32)],
    )(x, s1, s2, s4, s8, s16,
      t2[0], t2[1], t4[0], t4[1], t8[0], t8[1], t16[0], t16[1], deg,
      W1, b1, W2, b2, W3, b3, We, be, Wc, bc)


def kernel(x, edge_index, batch, W1, b1, W2, b2, W3, b3, We, be, Wc, bc):
    pad = jnp.full((EPAD,), N, jnp.int32)
    src2 = jnp.concatenate([edge_index[0], pad]).reshape(EROWS, 128)
    dst2 = jnp.concatenate([edge_index[1], pad]).reshape(EROWS, 128)
    zer16 = jnp.zeros((NPT, 16), jnp.float32)
    ones16 = jnp.ones((NROWS, 16), jnp.float32)

    sc16 = _make_sc_pass(16)

    d0, d1 = sc16(ones16, src2, src2, zer16)
    inv, deg, s = _prep(x, d0, d1)

    snaps1 = {}
    for k in range(1, 17):
        p0, p1 = sc16(s, src2, dst2, zer16)
        s = _combine(s, p0, p1, inv)
        if k in (1, 2, 4, 8, 16):
            snaps1[k] = s

    ta, tb = _assemble(snaps1[1], snaps1[2], snaps1[4], snaps1[8])

    snaps2 = {}
    for k in range(1, 17):
        p0a, p1a = sc16(ta, src2, dst2, zer16)
        p0b, p1b = sc16(tb, src2, dst2, zer16)
        ta = _combine(ta, p0a, p1a, inv)
        tb = _combine(tb, p0b, p1b, inv)
        if k in (2, 4, 8, 16):
            snaps2[k] = (ta, tb)

    emb, out = _mlp(x, snaps1[1], snaps1[2], snaps1[4], snaps1[8], snaps1[16],
                    snaps2[2], snaps2[4], snaps2[8], snaps2[16], deg,
                    W1, b1, W2, b2, W3, b3, We, be, Wc, bc)
    return (emb, out)
